# baseline (speedup 1.0000x reference)
import jax
import jax.numpy as jnp
from jax import lax
from jax.experimental import pallas as pl
from jax.experimental.pallas import tpu as pltpu
from jax.experimental.pallas import tpu_sc as plsc

EMB = 64
SEQ = 201
BATCH_ROWS = 1024
VOCAB_ROWS = 1000002
TBLK = 16384
NBLK = -(-VOCAB_ROWS // TBLK)
ROWS_LIN = NBLK * TBLK
NUM_CORES = 2
NUM_SUBCORES = 16
NW = NUM_CORES * NUM_SUBCORES
SCALE = 8.0

HALF = TBLK // 2
_LAST_BLK = VOCAB_ROWS // HALF
_LOG2_HALF = HALF.bit_length() - 1

BB = 128
NTASK = SEQ * (BATCH_ROWS // BB)
BASE_T = NTASK // NW
EXTRA = NTASK - BASE_T * NW
NTILE3 = SEQ * (EMB // 8) * (BATCH_ROWS // BB)


def _linearize_body(a_ref, b_ref, o_ref):
    ya = jnp.transpose(a_ref[...], (1, 0))
    yb = jnp.transpose(b_ref[...], (1, 0))
    o_ref[...] = jnp.concatenate([ya, yb], axis=1) * SCALE


_linearize = pl.pallas_call(
    _linearize_body,
    grid=(NBLK,),
    in_specs=[
        pl.BlockSpec((EMB, HALF), lambda i: (0, jnp.minimum(2 * i, _LAST_BLK))),
        pl.BlockSpec((EMB, HALF), lambda i: (0, jnp.minimum(2 * i + 1, _LAST_BLK))),
    ],
    out_specs=pl.BlockSpec((HALF, 2 * EMB), lambda i: (i, 0)),
    out_shape=jax.ShapeDtypeStruct((NBLK * HALF, 2 * EMB), jnp.float32),
)


def _body(idx_hbm, table_hbm, pos_hbm, out_hbm,
          idx_slab, pos_v, r0, r1, o0, o1, g0, g1, s0, s1):
    rows = (r0, r1)
    ogs = (o0, o1)
    gsems = (g0, g1)
    ssems = (s0, s1)
    wid = lax.axis_index("s") * NUM_CORES + lax.axis_index("c")
    start = BASE_T * wid + jnp.minimum(wid, EXTRA)
    ntask = BASE_T + (wid < EXTRA).astype(jnp.int32)
    l0 = start // 8

    pltpu.sync_copy(idx_hbm.at[pl.ds(l0, 8)], idx_slab)
    pltpu.sync_copy(pos_hbm, pos_v)

    jvs = [lax.iota(jnp.int32, 16) + bc * 16 for bc in range(EMB // 8)]

    def task_coords(k):
        tau = start + k
        l = tau // 8
        tc = tau % 8
        return l, tc

    def fire_gather(k, slot):
        l, tc = task_coords(k)
        idx_ref = idx_slab.at[l - l0, pl.ds(tc * BB, BB)]
        return pltpu.async_copy(table_hbm.at[idx_ref], rows[slot], gsems[slot])

    def drain_gather(slot):
        pltpu.make_async_copy(
            table_hbm.at[pl.ds(0, BB)], rows[slot], gsems[slot]
        ).wait()

    def drain_writes(slot):
        for tr in range(EMB // 8):
            pltpu.make_async_copy(
                ogs[slot].at[pl.ds(tr * 8, 8)], out_hbm.at[0], ssems[slot]
            ).wait()

    def stage(k, slot):
        @pl.when(k < ntask)
        def _():
            l, tc = task_coords(k)
            drain_gather(slot)

            @pl.when(k >= 2)
            def _():
                drain_writes(slot)

            rbuf = rows[slot]
            obuf = ogs[slot]
            pbase = l * EMB

            @pl.loop(0, EMB)
            def _col(e):
                pv = plsc.load_gather(pos_v, [jnp.full((16,), 0, jnp.int32) + pbase + e])
                ev = jnp.full((16,), 0, jnp.int32) + e
                for bc in range(EMB // 8):
                    g = plsc.load_gather(rbuf, [jvs[bc], ev])
                    obuf[e, pl.ds(bc * 16, 16)] = g + pv

            t3 = l * EMB + tc
            for tr in range(EMB // 8):
                pltpu.async_copy(
                    obuf.at[pl.ds(tr * 8, 8)], out_hbm.at[t3 + tr * 8], ssems[slot]
                )

            @pl.when(k + 2 < ntask)
            def _():
                fire_gather(k + 2, slot)

    fire_gather(0, 0)
    fire_gather(1, 1)

    @pl.loop(0, (BASE_T + 2) // 2 * 2, step=2)
    def _pair(k):
        stage(k, 0)
        stage(k + 1, 1)

    drain_writes(0)
    drain_writes(1)


_encodings = pl.kernel(
    _body,
    out_type=jax.ShapeDtypeStruct((NTILE3, 8, BB), jnp.float32),
    mesh=plsc.VectorSubcoreMesh(core_axis_name="c", subcore_axis_name="s"),
    compiler_params=pltpu.CompilerParams(
        use_tc_tiling_on_sc=False, needs_layout_passes=False
    ),
    scratch_types=[
        pltpu.VMEM((8, BATCH_ROWS), jnp.int32),
        pltpu.VMEM((SEQ * EMB,), jnp.float32),
        pltpu.VMEM((BB, EMB), jnp.float32),
        pltpu.VMEM((BB, EMB), jnp.float32),
        pltpu.VMEM((EMB, BB), jnp.float32),
        pltpu.VMEM((EMB, BB), jnp.float32),
        pltpu.SemaphoreType.DMA,
        pltpu.SemaphoreType.DMA,
        pltpu.SemaphoreType.DMA,
        pltpu.SemaphoreType.DMA,
    ],
)


@jax.jit
def kernel(batch, table, pos_table):
    table_t = table.T
    lin = _linearize(table_t, table_t)
    table_lin = jnp.reshape(lin, (ROWS_LIN, EMB))
    t = batch.astype(jnp.int32)
    rt = (t & ~(TBLK - 1)) | ((t & (HALF - 1)) << 1) | ((t >> _LOG2_HALF) & 1)
    rt_t = jnp.pad(rt.T, ((0, 8 * ((SEQ + 7) // 8) - SEQ), (0, 0)))
    pos_flat = jnp.reshape(pos_table, (SEQ * EMB,))
    out3 = _encodings(rt_t, table_lin, pos_flat)
    x = jnp.reshape(out3, (SEQ, 8, 8, 8, BB))
    return jnp.reshape(jnp.transpose(x, (2, 4, 0, 1, 3)), (BATCH_ROWS, SEQ, EMB))


# speedup vs baseline: 1.5630x; 1.5630x over previous
import jax
import jax.numpy as jnp
from jax import lax
from jax.experimental import pallas as pl
from jax.experimental.pallas import tpu as pltpu
from jax.experimental.pallas import tpu_sc as plsc

EMB = 64
SEQ = 201
BATCH_ROWS = 1024
VOCAB_ROWS = 1000002
TBLK = 16384
NBLK = -(-VOCAB_ROWS // TBLK)
ROWS_LIN = NBLK * TBLK
NUM_CORES = 2
NUM_SUBCORES = 16
NW = NUM_CORES * NUM_SUBCORES
SCALE = 8.0

HALF = TBLK // 2
_LAST_BLK = VOCAB_ROWS // HALF
_LOG2_HALF = HALF.bit_length() - 1

BB = 128
NTASK = SEQ * (BATCH_ROWS // BB)
BASE_T = NTASK // NW
EXTRA = NTASK - BASE_T * NW
NTILE3 = SEQ * (EMB // 8) * (BATCH_ROWS // BB)


def _linearize_body(a_ref, b_ref, o_ref):
    ya = jnp.transpose(a_ref[...], (1, 0))
    yb = jnp.transpose(b_ref[...], (1, 0))
    o_ref[...] = jnp.concatenate([ya, yb], axis=1) * SCALE


_linearize = pl.pallas_call(
    _linearize_body,
    grid=(NBLK,),
    in_specs=[
        pl.BlockSpec((EMB, HALF), lambda i: (0, jnp.minimum(2 * i, _LAST_BLK))),
        pl.BlockSpec((EMB, HALF), lambda i: (0, jnp.minimum(2 * i + 1, _LAST_BLK))),
    ],
    out_specs=pl.BlockSpec((HALF, 2 * EMB), lambda i: (i, 0)),
    out_shape=jax.ShapeDtypeStruct((NBLK * HALF, 2 * EMB), jnp.float32),
)


def _body(idx_hbm, table_hbm, pos_hbm, out_hbm,
          idx_slab, pos_v, r0, r1, o0, o1, g0, g1, s0, s1):
    rows = (r0, r1)
    ogs = (o0, o1)
    gsems = (g0, g1)
    ssems = (s0, s1)
    wid = lax.axis_index("s") * NUM_CORES + lax.axis_index("c")
    start = BASE_T * wid + jnp.minimum(wid, EXTRA)
    ntask = BASE_T + (wid < EXTRA).astype(jnp.int32)
    l0 = start // 8

    pltpu.sync_copy(idx_hbm.at[pl.ds(l0, 8)], idx_slab)
    pltpu.sync_copy(pos_hbm, pos_v)

    jvs = [lax.iota(jnp.int32, 16) + c * 16 for c in range(EMB // 16)]

    def task_coords(k):
        tau = start + k
        l = tau // 8
        tc = tau % 8
        return l, tc

    def fire_gather(k, slot):
        l, tc = task_coords(k)
        idx_ref = idx_slab.at[l - l0, pl.ds(tc * BB, BB)]
        return pltpu.async_copy(table_hbm.at[idx_ref], rows[slot], gsems[slot])

    def drain_gather(slot):
        pltpu.make_async_copy(
            table_hbm.at[pl.ds(0, BB)], rows[slot], gsems[slot]
        ).wait()

    def drain_writes(slot):
        for tr in range(EMB // 8):
            pltpu.make_async_copy(
                ogs[slot].at[pl.ds(tr * 8, 8), pl.ds(0, BB)],
                out_hbm.at[0], ssems[slot],
            ).wait()

    def stage(k, slot):
        @pl.when(k < ntask)
        def _():
            l, tc = task_coords(k)
            drain_gather(slot)

            @pl.when(k >= 2)
            def _():
                drain_writes(slot)

            rbuf = rows[slot]
            obuf = ogs[slot]
            pbase = l * EMB
            pvs = [pos_v[pl.ds(pbase + c * 16, 16)] for c in range(EMB // 16)]

            @pl.loop(0, BB, unroll=2)
            def _row(j):
                jv = jnp.full((16,), 0, jnp.int32) + j
                for c in range(EMB // 16):
                    val = rbuf[j, pl.ds(c * 16, 16)] + pvs[c]
                    plsc.store_scatter(obuf, [jvs[c], jv], val)

            t3 = l * EMB + tc
            for tr in range(EMB // 8):
                pltpu.async_copy(
                    obuf.at[pl.ds(tr * 8, 8), pl.ds(0, BB)],
                    out_hbm.at[t3 + tr * 8], ssems[slot],
                )

            @pl.when(k + 2 < ntask)
            def _():
                fire_gather(k + 2, slot)

    fire_gather(0, 0)
    fire_gather(1, 1)

    @pl.loop(0, (BASE_T + 2) // 2 * 2, step=2)
    def _pair(k):
        stage(k, 0)
        stage(k + 1, 1)

    drain_writes(0)
    drain_writes(1)


_encodings = pl.kernel(
    _body,
    out_type=jax.ShapeDtypeStruct((NTILE3, 8, BB), jnp.float32),
    mesh=plsc.VectorSubcoreMesh(core_axis_name="c", subcore_axis_name="s"),
    compiler_params=pltpu.CompilerParams(
        use_tc_tiling_on_sc=False, needs_layout_passes=False
    ),
    scratch_types=[
        pltpu.VMEM((8, BATCH_ROWS), jnp.int32),
        pltpu.VMEM((SEQ * EMB,), jnp.float32),
        pltpu.VMEM((BB, EMB), jnp.float32),
        pltpu.VMEM((BB, EMB), jnp.float32),
        pltpu.VMEM((EMB, BB + 1), jnp.float32),
        pltpu.VMEM((EMB, BB + 1), jnp.float32),
        pltpu.SemaphoreType.DMA,
        pltpu.SemaphoreType.DMA,
        pltpu.SemaphoreType.DMA,
        pltpu.SemaphoreType.DMA,
    ],
)


@jax.jit
def kernel(batch, table, pos_table):
    table_t = table.T
    lin = _linearize(table_t, table_t)
    table_lin = jnp.reshape(lin, (ROWS_LIN, EMB))
    t = batch.astype(jnp.int32)
    rt = (t & ~(TBLK - 1)) | ((t & (HALF - 1)) << 1) | ((t >> _LOG2_HALF) & 1)
    rt_t = jnp.pad(rt.T, ((0, 8 * ((SEQ + 7) // 8) - SEQ), (0, 0)))
    pos_flat = jnp.reshape(pos_table, (SEQ * EMB,))
    out3 = _encodings(rt_t, table_lin, pos_flat)
    x = jnp.reshape(out3, (SEQ, 8, 8, 8, BB))
    return jnp.reshape(jnp.transpose(x, (2, 4, 0, 1, 3)), (BATCH_ROWS, SEQ, EMB))


# speedup vs baseline: 1.6230x; 1.0384x over previous
import jax
import jax.numpy as jnp
from jax import lax
from jax.experimental import pallas as pl
from jax.experimental.pallas import tpu as pltpu
from jax.experimental.pallas import tpu_sc as plsc

EMB = 64
SEQ = 201
BATCH_ROWS = 1024
VOCAB_ROWS = 1000002
TBLK = 32768
NBLK = -(-VOCAB_ROWS // TBLK)
ROWS_LIN = NBLK * TBLK
NUM_CORES = 2
NUM_SUBCORES = 16
NW = NUM_CORES * NUM_SUBCORES
SCALE = 8.0

HALF = TBLK // 2
_LAST_BLK = VOCAB_ROWS // HALF
_LOG2_HALF = HALF.bit_length() - 1

BB = 128
NTASK = SEQ * (BATCH_ROWS // BB)
BASE_T = NTASK // NW
EXTRA = NTASK - BASE_T * NW
NTILE3 = SEQ * (EMB // 8) * (BATCH_ROWS // BB)


def _linearize_body(a_ref, b_ref, o_ref):
    ya = jnp.transpose(a_ref[...], (1, 0))
    yb = jnp.transpose(b_ref[...], (1, 0))
    o_ref[...] = jnp.concatenate([ya, yb], axis=1) * SCALE


_linearize = pl.pallas_call(
    _linearize_body,
    grid=(NBLK,),
    in_specs=[
        pl.BlockSpec((EMB, HALF), lambda i: (0, jnp.minimum(2 * i, _LAST_BLK))),
        pl.BlockSpec((EMB, HALF), lambda i: (0, jnp.minimum(2 * i + 1, _LAST_BLK))),
    ],
    out_specs=pl.BlockSpec((HALF, 2 * EMB), lambda i: (i, 0)),
    out_shape=jax.ShapeDtypeStruct((NBLK * HALF, 2 * EMB), jnp.float32),
)


def _body(idx_hbm, table_hbm, pos_hbm, out_hbm,
          idx_slab, pos_v, r0, r1, o0, o1, g0, g1, s0, s1):
    rows = (r0, r1)
    ogs = (o0, o1)
    gsems = (g0, g1)
    ssems = (s0, s1)
    wid = lax.axis_index("s") * NUM_CORES + lax.axis_index("c")
    start = BASE_T * wid + jnp.minimum(wid, EXTRA)
    ntask = BASE_T + (wid < EXTRA).astype(jnp.int32)
    l0 = start // 8

    pltpu.sync_copy(idx_hbm.at[pl.ds(l0, 8)], idx_slab)
    pltpu.sync_copy(pos_hbm, pos_v)

    jvs = [lax.iota(jnp.int32, 16) + c * 16 for c in range(EMB // 16)]

    def task_coords(k):
        tau = start + k
        l = tau // 8
        tc = tau % 8
        return l, tc

    def fire_gather(k, slot):
        l, tc = task_coords(k)
        idx_ref = idx_slab.at[l - l0, pl.ds(tc * BB, BB)]
        return pltpu.async_copy(table_hbm.at[idx_ref], rows[slot], gsems[slot])

    def drain_gather(slot):
        pltpu.make_async_copy(
            table_hbm.at[pl.ds(0, BB)], rows[slot], gsems[slot]
        ).wait()

    def drain_writes(slot):
        for tr in range(EMB // 8):
            pltpu.make_async_copy(
                ogs[slot].at[pl.ds(tr * 8, 8), pl.ds(0, BB)],
                out_hbm.at[0], ssems[slot],
            ).wait()

    def stage(k, slot):
        @pl.when(k < ntask)
        def _():
            l, tc = task_coords(k)
            drain_gather(slot)

            @pl.when(k >= 2)
            def _():
                drain_writes(slot)

            rbuf = rows[slot]
            obuf = ogs[slot]
            pbase = l * EMB
            pvs = [pos_v[pl.ds(pbase + c * 16, 16)] for c in range(EMB // 16)]

            @pl.loop(0, BB, unroll=2)
            def _row(j):
                jv = jnp.full((16,), 0, jnp.int32) + j
                for c in range(EMB // 16):
                    val = rbuf[j, pl.ds(c * 16, 16)] + pvs[c]
                    plsc.store_scatter(obuf, [jvs[c], jv], val)

            t3 = l * EMB + tc
            for tr in range(EMB // 8):
                pltpu.async_copy(
                    obuf.at[pl.ds(tr * 8, 8), pl.ds(0, BB)],
                    out_hbm.at[t3 + tr * 8], ssems[slot],
                )

            @pl.when(k + 2 < ntask)
            def _():
                fire_gather(k + 2, slot)

    fire_gather(0, 0)
    fire_gather(1, 1)

    @pl.loop(0, (BASE_T + 2) // 2 * 2, step=2)
    def _pair(k):
        stage(k, 0)
        stage(k + 1, 1)

    drain_writes(0)
    drain_writes(1)


_encodings = pl.kernel(
    _body,
    out_type=jax.ShapeDtypeStruct((NTILE3, 8, BB), jnp.float32),
    mesh=plsc.VectorSubcoreMesh(core_axis_name="c", subcore_axis_name="s"),
    compiler_params=pltpu.CompilerParams(
        use_tc_tiling_on_sc=False, needs_layout_passes=False
    ),
    scratch_types=[
        pltpu.VMEM((8, BATCH_ROWS), jnp.int32),
        pltpu.VMEM((SEQ * EMB,), jnp.float32),
        pltpu.VMEM((BB, EMB), jnp.float32),
        pltpu.VMEM((BB, EMB), jnp.float32),
        pltpu.VMEM((EMB, BB + 1), jnp.float32),
        pltpu.VMEM((EMB, BB + 1), jnp.float32),
        pltpu.SemaphoreType.DMA,
        pltpu.SemaphoreType.DMA,
        pltpu.SemaphoreType.DMA,
        pltpu.SemaphoreType.DMA,
    ],
)


@jax.jit
def kernel(batch, table, pos_table):
    table_t = table.T
    lin = _linearize(table_t, table_t)
    table_lin = jnp.reshape(lin, (ROWS_LIN, EMB))
    t = batch.astype(jnp.int32)
    rt = (t & ~(TBLK - 1)) | ((t & (HALF - 1)) << 1) | ((t >> _LOG2_HALF) & 1)
    rt_t = jnp.pad(rt.T, ((0, 8 * ((SEQ + 7) // 8) - SEQ), (0, 0)))
    pos_flat = jnp.reshape(pos_table, (SEQ * EMB,))
    out3 = _encodings(rt_t, table_lin, pos_flat)
    x = jnp.reshape(out3, (SEQ, 8, 8, 8, BB))
    return jnp.reshape(jnp.transpose(x, (2, 4, 0, 1, 3)), (BATCH_ROWS, SEQ, EMB))


# speedup vs baseline: 1.6346x; 1.0071x over previous
import jax
import jax.numpy as jnp
from jax import lax
from jax.experimental import pallas as pl
from jax.experimental.pallas import tpu as pltpu
from jax.experimental.pallas import tpu_sc as plsc

EMB = 64
SEQ = 201
BATCH_ROWS = 1024
VOCAB_ROWS = 1000002
TBLK = 32768
NBLK = -(-VOCAB_ROWS // TBLK)
ROWS_LIN = NBLK * TBLK
NUM_CORES = 2
NUM_SUBCORES = 16
NW = NUM_CORES * NUM_SUBCORES
SCALE = 8.0

HALF = TBLK // 2
_LAST_BLK = VOCAB_ROWS // HALF
_LOG2_HALF = HALF.bit_length() - 1

BB = 128
NTASK = SEQ * (BATCH_ROWS // BB)
BASE_T = NTASK // NW
EXTRA = NTASK - BASE_T * NW
NTILE3 = SEQ * (EMB // 8) * (BATCH_ROWS // BB)


def _linearize_body(a_ref, b_ref, o_ref):
    ya = jnp.transpose(a_ref[...], (1, 0))
    yb = jnp.transpose(b_ref[...], (1, 0))
    o_ref[...] = jnp.concatenate([ya, yb], axis=1) * SCALE


_linearize = pl.pallas_call(
    _linearize_body,
    grid=(NBLK,),
    in_specs=[
        pl.BlockSpec((EMB, HALF), lambda i: (0, jnp.minimum(2 * i, _LAST_BLK))),
        pl.BlockSpec((EMB, HALF), lambda i: (0, jnp.minimum(2 * i + 1, _LAST_BLK))),
    ],
    out_specs=pl.BlockSpec((HALF, 2 * EMB), lambda i: (i, 0)),
    out_shape=jax.ShapeDtypeStruct((NBLK * HALF, 2 * EMB), jnp.float32),
)


def _body(idx_hbm, table_hbm, pos_hbm, out_hbm,
          idx_slab, pos_v, r0, r1, o0, o1, g0, g1, s0, s1):
    rows = (r0, r1)
    ogs = (o0, o1)
    gsems = (g0, g1)
    ssems = (s0, s1)
    wid = lax.axis_index("s") * NUM_CORES + lax.axis_index("c")
    start = BASE_T * wid + jnp.minimum(wid, EXTRA)
    ntask = BASE_T + (wid < EXTRA).astype(jnp.int32)
    l0 = start // 8

    pltpu.sync_copy(idx_hbm.at[pl.ds(l0, 8)], idx_slab)
    pltpu.sync_copy(pos_hbm, pos_v)

    jvs = [lax.iota(jnp.int32, 16) + c * 16 for c in range(EMB // 16)]

    def task_coords(k):
        tau = start + k
        l = tau // 8
        tc = tau % 8
        return l, tc

    def fire_gather(k, slot):
        l, tc = task_coords(k)
        idx_ref = idx_slab.at[l - l0, pl.ds(tc * BB, BB)]
        return pltpu.async_copy(table_hbm.at[idx_ref], rows[slot], gsems[slot])

    def drain_gather(slot):
        pltpu.make_async_copy(
            table_hbm.at[pl.ds(0, BB)], rows[slot], gsems[slot]
        ).wait()

    def drain_writes(slot):
        for tr in range(EMB // 8):
            pltpu.make_async_copy(
                ogs[slot].at[pl.ds(tr * 8, 8), pl.ds(0, BB)],
                out_hbm.at[0], ssems[slot],
            ).wait()

    def stage(k, slot):
        @pl.when(k < ntask)
        def _():
            l, tc = task_coords(k)
            drain_gather(slot)

            @pl.when(k >= 2)
            def _():
                drain_writes(slot)

            rbuf = rows[slot]
            obuf = ogs[slot]
            pbase = l * EMB
            pvs = [pos_v[pl.ds(pbase + c * 16, 16)] for c in range(EMB // 16)]

            @pl.loop(0, BB, unroll=4)
            def _row(j):
                jv = jnp.full((16,), 0, jnp.int32) + j
                for c in range(EMB // 16):
                    val = rbuf[j, pl.ds(c * 16, 16)] + pvs[c]
                    plsc.store_scatter(obuf, [jvs[c], jv], val)

            t3 = l * EMB + tc
            for tr in range(EMB // 8):
                pltpu.async_copy(
                    obuf.at[pl.ds(tr * 8, 8), pl.ds(0, BB)],
                    out_hbm.at[t3 + tr * 8], ssems[slot],
                )

            @pl.when(k + 2 < ntask)
            def _():
                fire_gather(k + 2, slot)

    fire_gather(0, 0)
    fire_gather(1, 1)

    @pl.loop(0, (BASE_T + 2) // 2 * 2, step=2)
    def _pair(k):
        stage(k, 0)
        stage(k + 1, 1)

    drain_writes(0)
    drain_writes(1)


_encodings = pl.kernel(
    _body,
    out_type=jax.ShapeDtypeStruct((NTILE3, 8, BB), jnp.float32),
    mesh=plsc.VectorSubcoreMesh(core_axis_name="c", subcore_axis_name="s"),
    compiler_params=pltpu.CompilerParams(
        use_tc_tiling_on_sc=False, needs_layout_passes=False
    ),
    scratch_types=[
        pltpu.VMEM((8, BATCH_ROWS), jnp.int32),
        pltpu.VMEM((SEQ * EMB,), jnp.float32),
        pltpu.VMEM((BB, EMB), jnp.float32),
        pltpu.VMEM((BB, EMB), jnp.float32),
        pltpu.VMEM((EMB, BB + 1), jnp.float32),
        pltpu.VMEM((EMB, BB + 1), jnp.float32),
        pltpu.SemaphoreType.DMA,
        pltpu.SemaphoreType.DMA,
        pltpu.SemaphoreType.DMA,
        pltpu.SemaphoreType.DMA,
    ],
)


@jax.jit
def kernel(batch, table, pos_table):
    table_t = table.T
    lin = _linearize(table_t, table_t)
    table_lin = jnp.reshape(lin, (ROWS_LIN, EMB))
    t = batch.astype(jnp.int32)
    rt = (t & ~(TBLK - 1)) | ((t & (HALF - 1)) << 1) | ((t >> _LOG2_HALF) & 1)
    rt_t = jnp.pad(rt.T, ((0, 8 * ((SEQ + 7) // 8) - SEQ), (0, 0)))
    pos_flat = jnp.reshape(pos_table, (SEQ * EMB,))
    out3 = _encodings(rt_t, table_lin, pos_flat)
    x = jnp.reshape(out3, (SEQ, 8, 8, 8, BB))
    return jnp.reshape(jnp.transpose(x, (2, 4, 0, 1, 3)), (BATCH_ROWS, SEQ, EMB))
